# baseline (speedup 1.0000x reference)
import jax
import jax.numpy as jnp
from jax.experimental import pallas as pl
from jax.experimental.pallas import tpu as pltpu

_B = 256
_N = 4096
_DD = 256 * 256
_TILE = 4096
_NT = _DD // _TILE


def _opd_kernel(pos_ref, obs_t_ref, dic_ref, alpha_ref, s_ref, out_ref, c_ref):
    i = pl.program_id(0)

    @pl.when(i == 0)
    def _stage_a():
        px = pos_ref[:, 0:1]
        py = pos_ref[:, 1:2]
        ox = obs_t_ref[0:1, :]
        oy = obs_t_ref[1:2, :]
        d = (px - ox) ** 2 + (py - oy) ** 2
        md = jnp.min(d, axis=1, keepdims=True)
        iota = jax.lax.broadcasted_iota(jnp.int32, (_B, _N), 1)
        idx = jnp.min(jnp.where(d == md, iota, _N), axis=1, keepdims=True)
        onehot = (iota == idx).astype(jnp.float32)
        g = jnp.dot(onehot, dic_ref[...], preferred_element_type=jnp.float32)
        c_ref[...] = jnp.dot(g, alpha_ref[...], preferred_element_type=jnp.float32)

    out_ref[...] = jnp.dot(c_ref[...], s_ref[...],
                           preferred_element_type=jnp.float32)


def kernel(positions, obs_pos, poly_dic, graph_dic, S_poly, S_graph,
           alpha_poly, alpha_graph):
    pe, pf = alpha_poly.shape
    ge, gf = alpha_graph.shape
    k = pf + gf

    dics = jnp.concatenate([poly_dic, graph_dic], axis=1)
    alpha = jnp.zeros((pe + ge, k), jnp.float32)
    alpha = alpha.at[:pe, :pf].set(alpha_poly)
    alpha = alpha.at[pe:, pf:].set(alpha_graph)
    s_cat = jnp.concatenate(
        [S_poly.reshape(pf, _DD), S_graph.reshape(gf, _DD)], axis=0)
    obs_t = obs_pos.T

    out = pl.pallas_call(
        _opd_kernel,
        grid=(_NT,),
        in_specs=[
            pl.BlockSpec((_B, 2), lambda i: (0, 0)),
            pl.BlockSpec((2, _N), lambda i: (0, 0)),
            pl.BlockSpec(dics.shape, lambda i: (0, 0)),
            pl.BlockSpec(alpha.shape, lambda i: (0, 0)),
            pl.BlockSpec((k, _TILE), lambda i: (0, i)),
        ],
        out_specs=pl.BlockSpec((_B, _TILE), lambda i: (0, i)),
        out_shape=jax.ShapeDtypeStruct((_B, _DD), jnp.float32),
        scratch_shapes=[pltpu.VMEM((_B, k), jnp.float32)],
    )(positions, obs_t, dics, alpha, s_cat)

    opd_maps = out.reshape(_B, 256, 256)
    return (opd_maps, alpha_graph)


# speedup vs baseline: 2.6591x; 2.6591x over previous
import jax
import jax.numpy as jnp
from jax.experimental import pallas as pl
from jax.experimental.pallas import tpu as pltpu

_B = 256
_N = 4096
_D = 256
_M = 16
_NT = _D // _M


def _opd_kernel(pos_ref, obs_t_ref, dic_ref, alpha_ref, s_ref, out_ref, c_ref):
    i = pl.program_id(0)

    @pl.when(i == 0)
    def _stage_a():
        px = pos_ref[:, 0:1]
        py = pos_ref[:, 1:2]
        ox = obs_t_ref[0:1, :]
        oy = obs_t_ref[1:2, :]
        d = (px - ox) ** 2 + (py - oy) ** 2
        md = jnp.min(d, axis=1, keepdims=True)
        iota = jax.lax.broadcasted_iota(jnp.int32, (_B, _N), 1)
        idx = jnp.min(jnp.where(d == md, iota, _N), axis=1, keepdims=True)
        onehot = (iota == idx).astype(jnp.float32)
        g = jnp.dot(onehot, dic_ref[...], preferred_element_type=jnp.float32)
        c_ref[...] = jnp.dot(g, alpha_ref[...], preferred_element_type=jnp.float32)

    k = alpha_ref.shape[1]
    s2 = s_ref[...].reshape(k, _M * _D)
    r = jnp.dot(c_ref[...], s2, preferred_element_type=jnp.float32)
    out_ref[...] = r.reshape(_B, _M, _D)


def kernel(positions, obs_pos, poly_dic, graph_dic, S_poly, S_graph,
           alpha_poly, alpha_graph):
    pe, pf = alpha_poly.shape
    ge, gf = alpha_graph.shape
    k = pf + gf

    dics = jnp.concatenate([poly_dic, graph_dic], axis=1)
    alpha = jnp.zeros((pe + ge, k), jnp.float32)
    alpha = alpha.at[:pe, :pf].set(alpha_poly)
    alpha = alpha.at[pe:, pf:].set(alpha_graph)
    s_cat = jnp.concatenate([S_poly, S_graph], axis=0)
    obs_t = obs_pos.T

    opd_maps = pl.pallas_call(
        _opd_kernel,
        grid=(_NT,),
        in_specs=[
            pl.BlockSpec((_B, 2), lambda i: (0, 0)),
            pl.BlockSpec((2, _N), lambda i: (0, 0)),
            pl.BlockSpec(dics.shape, lambda i: (0, 0)),
            pl.BlockSpec(alpha.shape, lambda i: (0, 0)),
            pl.BlockSpec((k, _M, _D), lambda i: (0, i, 0)),
        ],
        out_specs=pl.BlockSpec((_B, _M, _D), lambda i: (0, i, 0)),
        out_shape=jax.ShapeDtypeStruct((_B, _D, _D), jnp.float32),
        scratch_shapes=[pltpu.VMEM((_B, k), jnp.float32)],
    )(positions, obs_t, dics, alpha, s_cat)

    return (opd_maps, alpha_graph)


# speedup vs baseline: 2.7960x; 1.0515x over previous
import jax
import jax.numpy as jnp
from jax.experimental import pallas as pl
from jax.experimental.pallas import tpu as pltpu

_B = 256
_N = 4096
_D = 256
_M = 32
_NT = _D // _M


def _opd_kernel(pos_ref, obs_t_ref, dic_ref, alpha_ref, s_ref, out_ref, c_ref):
    i = pl.program_id(0)

    @pl.when(i == 0)
    def _stage_a():
        px = pos_ref[:, 0:1]
        py = pos_ref[:, 1:2]
        ox = obs_t_ref[0:1, :]
        oy = obs_t_ref[1:2, :]
        d = (px - ox) ** 2 + (py - oy) ** 2
        md = jnp.min(d, axis=1, keepdims=True)
        iota = jax.lax.broadcasted_iota(jnp.int32, (_B, _N), 1)
        idx = jnp.min(jnp.where(d == md, iota, _N), axis=1, keepdims=True)
        onehot = (iota == idx).astype(jnp.float32)
        g = jnp.dot(onehot, dic_ref[...], preferred_element_type=jnp.float32)
        c_ref[...] = jnp.dot(g, alpha_ref[...], preferred_element_type=jnp.float32)

    k = alpha_ref.shape[1]
    s2 = s_ref[...].reshape(k, _M * _D)
    r = jnp.dot(c_ref[...], s2, preferred_element_type=jnp.float32)
    out_ref[...] = r.reshape(_B, _M, _D)


def kernel(positions, obs_pos, poly_dic, graph_dic, S_poly, S_graph,
           alpha_poly, alpha_graph):
    pe, pf = alpha_poly.shape
    ge, gf = alpha_graph.shape
    k = pf + gf

    dics = jnp.concatenate([poly_dic, graph_dic], axis=1)
    alpha = jnp.zeros((pe + ge, k), jnp.float32)
    alpha = alpha.at[:pe, :pf].set(alpha_poly)
    alpha = alpha.at[pe:, pf:].set(alpha_graph)
    s_cat = jnp.concatenate([S_poly, S_graph], axis=0)
    obs_t = obs_pos.T

    opd_maps = pl.pallas_call(
        _opd_kernel,
        grid=(_NT,),
        in_specs=[
            pl.BlockSpec((_B, 2), lambda i: (0, 0)),
            pl.BlockSpec((2, _N), lambda i: (0, 0)),
            pl.BlockSpec(dics.shape, lambda i: (0, 0)),
            pl.BlockSpec(alpha.shape, lambda i: (0, 0)),
            pl.BlockSpec((k, _M, _D), lambda i: (0, i, 0)),
        ],
        out_specs=pl.BlockSpec((_B, _M, _D), lambda i: (0, i, 0)),
        out_shape=jax.ShapeDtypeStruct((_B, _D, _D), jnp.float32),
        scratch_shapes=[pltpu.VMEM((_B, k), jnp.float32)],
    )(positions, obs_t, dics, alpha, s_cat)

    return (opd_maps, alpha_graph)
